# baseline (speedup 1.0000x reference)
import functools

import jax
import jax.numpy as jnp
from jax import lax
from jax.experimental import pallas as pl
from jax.experimental.pallas import tpu as pltpu
from jax.experimental.pallas import tpu_sc as plsc

NC = 2
NS = 16
NW = NC * NS
LANES = 16
CHUNK = 80
RING = 4
NSTAGE = 2


def _sc_aggregate(feat_packed, eflat, wgt, n_pad):
    n_nodes, dp = feat_packed.shape
    d = dp * 2
    n_edges = eflat.shape[0] // 2
    assert n_edges % NW == 0
    e_w = n_edges // NW
    assert e_w % CHUNK == 0
    n_chunks = e_w // CHUNK
    assert n_chunks >= RING
    n_iters = -(-(n_chunks + 2) // RING) * RING
    assert n_iters >= n_chunks + 2
    assert n_pad % (NS * 8) == 0
    rows_w = n_pad // NS
    assert rows_w % CHUNK == 0
    d_vecs = d // LANES
    dp_vecs = dp // LANES
    groups = CHUNK // LANES

    mesh = plsc.VectorSubcoreMesh(core_axis_name="c", subcore_axis_name="s")

    @functools.partial(
        pl.kernel,
        out_type=jax.ShapeDtypeStruct((NC, n_pad, d), jnp.float32),
        mesh=mesh,
        compiler_params=pltpu.CompilerParams(use_tc_tiling_on_sc=False),
        scratch_types=[
            pltpu.VMEM((RING, CHUNK), jnp.int32),
            pltpu.VMEM((RING, CHUNK), jnp.int32),
            pltpu.VMEM((RING, CHUNK), jnp.float32),
            pltpu.VMEM((RING, CHUNK), jnp.int32),
            pltpu.VMEM((RING, CHUNK), jnp.float32),
            pltpu.VMEM((RING, CHUNK, 64), jnp.int32),
            pltpu.VMEM((NSTAGE, CHUNK, 128), jnp.float32),
            pltpu.VMEM_SHARED((n_pad, 128), jnp.float32),
            [pltpu.SemaphoreType.DMA] * RING,
            [pltpu.SemaphoreType.DMA] * RING,
            [pltpu.SemaphoreType.DMA] * RING,
        ],
    )
    def agg(feat_hbm, eflat_hbm, wgt_hbm, out_hbm,
            src_f, dst_f, w_f, sdst, sw, rows, stag, acc,
            sem_i, sem_g, sem_s):
        c = lax.axis_index("c")
        s = lax.axis_index("s")
        wid = c * NS + s
        e0 = wid * e_w

        def idx_start(i, r):
            base = e0 + i * CHUNK
            pltpu.async_copy(eflat_hbm.at[pl.ds(n_edges + base, CHUNK)], src_f.at[r], sem_i[r])
            pltpu.async_copy(eflat_hbm.at[pl.ds(base, CHUNK)], dst_f.at[r], sem_i[r])
            pltpu.async_copy(wgt_hbm.at[pl.ds(base, CHUNK)], w_f.at[r], sem_i[r])

        def idx_wait(r):
            pltpu.make_async_copy(eflat_hbm.at[pl.ds(0, CHUNK)], src_f.at[r], sem_i[r]).wait()
            pltpu.make_async_copy(eflat_hbm.at[pl.ds(0, CHUNK)], dst_f.at[r], sem_i[r]).wait()
            pltpu.make_async_copy(wgt_hbm.at[pl.ds(0, CHUNK)], w_f.at[r], sem_i[r]).wait()

        def gather_start(r):
            pltpu.async_copy(feat_hbm.at[src_f.at[r]], rows.at[r], sem_g[r])

        def gather_wait(r):
            pltpu.make_async_copy(feat_hbm.at[src_f.at[r]], rows.at[r], sem_g[r]).wait()

        def scat_start(r, p2):
            pltpu.async_copy(stag.at[p2], acc.at[sdst.at[r]], sem_s[r], add=True)

        def scat_wait(r, p2):
            pltpu.make_async_copy(stag.at[p2], acc.at[sdst.at[r]], sem_s[r]).wait()

        def stash_idx(r):
            for g in range(groups):
                sl = pl.ds(g * LANES, LANES)
                sdst[r, sl] = dst_f[r, sl]
                sw[r, sl] = w_f[r, sl]

        hi_mask = jnp.full((LANES,), -65536, jnp.int32)

        def compute(r, p2):
            def group_body(g, carry):
                wv16 = sw[r, pl.ds(g * LANES, LANES)]
                for e16 in range(LANES):
                    wbc = lax.gather(
                        wv16,
                        jnp.full((LANES, 1), e16, jnp.int32),
                        dimension_numbers=lax.GatherDimensionNumbers(
                            offset_dims=(), collapsed_slice_dims=(0,),
                            start_index_map=(0,)),
                        slice_sizes=(1,),
                        mode=lax.GatherScatterMode.PROMISE_IN_BOUNDS)
                    e = g * LANES + e16
                    for j in range(dp_vecs):
                        xi = rows[r, e, pl.ds(j * LANES, LANES)]
                        ev = lax.bitcast_convert_type(
                            lax.shift_left(xi, 16), jnp.float32)
                        od = lax.bitcast_convert_type(
                            lax.bitwise_and(xi, hi_mask), jnp.float32)
                        stag[p2, e, pl.ds(j * 2 * LANES, LANES)] = ev * wbc
                        stag[p2, e, pl.ds((j * 2 + 1) * LANES, LANES)] = od * wbc
                return carry
            lax.fori_loop(0, groups, group_body, 0)

        idx_start(0, 0)
        idx_start(1, 1)
        idx_start(2, 2)

        def zfill(i, carry):
            for j in range(d_vecs):
                stag[0, i, pl.ds(j * LANES, LANES)] = jnp.zeros((LANES,), jnp.float32)
            return carry
        lax.fori_loop(0, CHUNK, zfill, 0)
        r0 = s * rows_w

        def zcopy(i, carry):
            pltpu.sync_copy(stag.at[0], acc.at[pl.ds(r0 + i * CHUNK, CHUNK), :])
            return carry
        lax.fori_loop(0, rows_w // CHUNK, zcopy, 0)

        idx_wait(0)
        gather_start(0)
        idx_wait(1)
        gather_start(1)
        plsc.subcore_barrier()

        def step(k, carry):
            for j in range(RING):
                i = k * RING + j
                live = i < n_chunks

                @pl.when(live)
                def _():
                    gather_wait(j)
                    stash_idx(j)

                @pl.when(i + 3 < n_chunks)
                def _():
                    idx_start(i + 3, (j + 3) % RING)

                @pl.when(jnp.logical_and(i >= 2, i - 2 < n_chunks))
                def _():
                    scat_wait((j + 2) % RING, j % NSTAGE)

                @pl.when(i + 2 < n_chunks)
                def _():
                    idx_wait((j + 2) % RING)
                    gather_start((j + 2) % RING)

                @pl.when(live)
                def _():
                    compute(j, j % NSTAGE)
                    scat_start(j, j % NSTAGE)
            return carry
        lax.fori_loop(0, n_iters // RING, step, 0)
        plsc.subcore_barrier()

        pltpu.sync_copy(acc.at[pl.ds(r0, rows_w), :],
                        out_hbm.at[c, pl.ds(r0, rows_w), :])

    return agg(feat_packed, eflat, wgt)


def _tc_combine_matmul_relu(partials, W_perm, n_nodes):
    _, n_pad, d = partials.shape
    d_out = W_perm.shape[1]
    blk = 1000
    assert n_nodes % blk == 0

    def body(p_ref, w_ref, o_ref):
        pp = p_ref[0] + p_ref[1]
        acc = jnp.dot(pp, w_ref[...], preferred_element_type=jnp.float32,
                      precision=lax.Precision.HIGHEST)
        o_ref[...] = jnp.maximum(acc, 0.0)

    return pl.pallas_call(
        body,
        grid=(n_nodes // blk,),
        in_specs=[
            pl.BlockSpec((NC, blk, d), lambda i: (0, i, 0)),
            pl.BlockSpec((d, d_out), lambda i: (0, 0)),
        ],
        out_specs=pl.BlockSpec((blk, d_out), lambda i: (i, 0)),
        out_shape=jax.ShapeDtypeStruct((n_nodes, d_out), jnp.float32),
    )(partials, W_perm)


def kernel(features, edge_index, edge_weight, W):
    n_nodes, d = features.shape
    n_pad = 10240
    eflat = edge_index.astype(jnp.int32).reshape(-1)
    wgt = edge_weight.astype(jnp.float32)
    feat_packed = lax.bitcast_convert_type(
        features.astype(jnp.bfloat16).reshape(n_nodes, d // 2, 2), jnp.int32)
    W_perm = W.reshape(d // 32, 16, 2, -1).transpose(0, 2, 1, 3).reshape(d, -1)
    partials = _sc_aggregate(feat_packed, eflat, wgt, n_pad)
    return _tc_combine_matmul_relu(partials, W_perm, n_nodes)


# speedup vs baseline: 1.9472x; 1.9472x over previous
import functools

import jax
import jax.numpy as jnp
from jax import lax
from jax.experimental import pallas as pl
from jax.experimental.pallas import tpu as pltpu
from jax.experimental.pallas import tpu_sc as plsc

NC = 2
NS = 16
NW = NC * NS
LANES = 16
CHUNK = 80
RING = 4
NSTAGE = 2


def _sc_aggregate(feat_packed, eflat, wgt, n_pad):
    n_nodes, dp = feat_packed.shape
    d = dp * 2
    n_edges = eflat.shape[0] // 2
    assert n_edges % NW == 0
    e_w = n_edges // NW
    assert e_w % CHUNK == 0
    n_chunks = e_w // CHUNK
    assert n_chunks >= RING
    n_iters = -(-(n_chunks + 2) // RING) * RING
    assert n_iters >= n_chunks + 2
    assert n_pad % (NS * 8) == 0
    rows_w = n_pad // NS
    assert rows_w % CHUNK == 0
    d_vecs = d // LANES
    dp_vecs = dp // LANES
    groups = CHUNK // LANES

    mesh = plsc.VectorSubcoreMesh(core_axis_name="c", subcore_axis_name="s")

    @functools.partial(
        pl.kernel,
        out_type=jax.ShapeDtypeStruct((NC, n_pad, d), jnp.float32),
        mesh=mesh,
        compiler_params=pltpu.CompilerParams(use_tc_tiling_on_sc=False),
        scratch_types=[
            pltpu.VMEM((RING, CHUNK), jnp.int32),
            pltpu.VMEM((RING, CHUNK), jnp.int32),
            pltpu.VMEM((RING, CHUNK), jnp.float32),
            pltpu.VMEM((RING, CHUNK), jnp.int32),
            pltpu.VMEM((RING, CHUNK), jnp.float32),
            pltpu.VMEM((RING, CHUNK, 64), jnp.int32),
            pltpu.VMEM((NSTAGE, CHUNK, 128), jnp.float32),
            pltpu.VMEM_SHARED((n_pad, 128), jnp.float32),
            [pltpu.SemaphoreType.DMA] * RING,
            [pltpu.SemaphoreType.DMA] * RING,
            [pltpu.SemaphoreType.DMA] * RING,
        ],
    )
    def agg(feat_hbm, eflat_hbm, wgt_hbm, out_hbm,
            src_f, dst_f, w_f, sdst, sw, rows, stag, acc,
            sem_i, sem_g, sem_s):
        c = lax.axis_index("c")
        s = lax.axis_index("s")
        wid = c * NS + s
        e0 = wid * e_w

        def idx_start(i, r):
            base = e0 + i * CHUNK
            pltpu.async_copy(eflat_hbm.at[pl.ds(n_edges + base, CHUNK)], src_f.at[r], sem_i[r])
            pltpu.async_copy(eflat_hbm.at[pl.ds(base, CHUNK)], dst_f.at[r], sem_i[r])
            pltpu.async_copy(wgt_hbm.at[pl.ds(base, CHUNK)], w_f.at[r], sem_i[r])

        def idx_wait(r):
            pltpu.make_async_copy(eflat_hbm.at[pl.ds(0, CHUNK)], src_f.at[r], sem_i[r]).wait()
            pltpu.make_async_copy(eflat_hbm.at[pl.ds(0, CHUNK)], dst_f.at[r], sem_i[r]).wait()
            pltpu.make_async_copy(wgt_hbm.at[pl.ds(0, CHUNK)], w_f.at[r], sem_i[r]).wait()

        def gather_start(r):
            pltpu.async_copy(feat_hbm.at[src_f.at[r]], rows.at[r], sem_g[r])

        def gather_wait(r):
            pltpu.make_async_copy(feat_hbm.at[src_f.at[r]], rows.at[r], sem_g[r]).wait()

        def scat_start(r, p2):
            pltpu.async_copy(stag.at[p2], acc.at[sdst.at[r]], sem_s[r], add=True)

        def scat_wait(r, p2):
            pltpu.make_async_copy(stag.at[p2], acc.at[sdst.at[r]], sem_s[r]).wait()

        def stash_idx(r):
            for g in range(groups):
                sl = pl.ds(g * LANES, LANES)
                sdst[r, sl] = dst_f[r, sl]
                sw[r, sl] = w_f[r, sl]

        hi_mask = jnp.full((LANES,), -65536, jnp.int32)

        def compute(r, p2):
            def group_body(g, carry):
                wv16 = sw[r, pl.ds(g * LANES, LANES)]
                for quad in range(LANES // 4):
                    xs = [[rows[r, g * LANES + quad * 4 + t,
                                pl.ds(j * LANES, LANES)]
                           for j in range(dp_vecs)] for t in range(4)]
                    for t in range(4):
                        e16 = quad * 4 + t
                        wbc = lax.gather(
                            wv16,
                            jnp.full((LANES, 1), e16, jnp.int32),
                            dimension_numbers=lax.GatherDimensionNumbers(
                                offset_dims=(), collapsed_slice_dims=(0,),
                                start_index_map=(0,)),
                            slice_sizes=(1,),
                            mode=lax.GatherScatterMode.PROMISE_IN_BOUNDS)
                        e = g * LANES + e16
                        for j in range(dp_vecs):
                            xi = xs[t][j]
                            ev = lax.bitcast_convert_type(
                                lax.shift_left(xi, 16), jnp.float32)
                            od = lax.bitcast_convert_type(
                                lax.bitwise_and(xi, hi_mask), jnp.float32)
                            stag[p2, e, pl.ds(j * 2 * LANES, LANES)] = ev * wbc
                            stag[p2, e, pl.ds((j * 2 + 1) * LANES, LANES)] = od * wbc
                return carry
            lax.fori_loop(0, groups, group_body, 0)

        idx_start(0, 0)
        idx_start(1, 1)
        idx_start(2, 2)

        def zfill(i, carry):
            for j in range(d_vecs):
                stag[0, i, pl.ds(j * LANES, LANES)] = jnp.zeros((LANES,), jnp.float32)
            return carry
        lax.fori_loop(0, CHUNK, zfill, 0)
        r0 = s * rows_w

        def zcopy(i, carry):
            pltpu.sync_copy(stag.at[0], acc.at[pl.ds(r0 + i * CHUNK, CHUNK), :])
            return carry
        lax.fori_loop(0, rows_w // CHUNK, zcopy, 0)

        idx_wait(0)
        gather_start(0)
        idx_wait(1)
        gather_start(1)
        plsc.subcore_barrier()

        def step(k, carry):
            for j in range(RING):
                i = k * RING + j
                live = i < n_chunks

                @pl.when(live)
                def _():
                    gather_wait(j)
                    stash_idx(j)

                @pl.when(i + 3 < n_chunks)
                def _():
                    idx_start(i + 3, (j + 3) % RING)

                @pl.when(jnp.logical_and(i >= 2, i - 2 < n_chunks))
                def _():
                    scat_wait((j + 2) % RING, j % NSTAGE)

                @pl.when(i + 2 < n_chunks)
                def _():
                    idx_wait((j + 2) % RING)
                    gather_start((j + 2) % RING)

                @pl.when(live)
                def _():
                    compute(j, j % NSTAGE)
                    scat_start(j, j % NSTAGE)
            return carry
        lax.fori_loop(0, n_iters // RING, step, 0)
        plsc.subcore_barrier()

        pltpu.sync_copy(acc.at[pl.ds(r0, rows_w), :],
                        out_hbm.at[c, pl.ds(r0, rows_w), :])

    return agg(feat_packed, eflat, wgt)


def _tc_combine_matmul_relu(partials, W_perm, n_nodes):
    _, n_pad, d = partials.shape
    d_out = W_perm.shape[1]
    blk = 1000
    assert n_nodes % blk == 0

    def body(p_ref, w_ref, o_ref):
        pp = p_ref[0] + p_ref[1]
        acc = jnp.dot(pp, w_ref[...], preferred_element_type=jnp.float32,
                      precision=lax.Precision.HIGHEST)
        o_ref[...] = jnp.maximum(acc, 0.0)

    return pl.pallas_call(
        body,
        grid=(n_nodes // blk,),
        in_specs=[
            pl.BlockSpec((NC, blk, d), lambda i: (0, i, 0)),
            pl.BlockSpec((d, d_out), lambda i: (0, 0)),
        ],
        out_specs=pl.BlockSpec((blk, d_out), lambda i: (i, 0)),
        out_shape=jax.ShapeDtypeStruct((n_nodes, d_out), jnp.float32),
    )(partials, W_perm)


def kernel(features, edge_index, edge_weight, W):
    n_nodes, d = features.shape
    n_pad = 10240
    eflat = edge_index.astype(jnp.int32).reshape(-1)
    wgt = edge_weight.astype(jnp.float32)
    feat_packed = lax.bitcast_convert_type(
        features.astype(jnp.bfloat16).reshape(n_nodes, d // 2, 2), jnp.int32)
    W_perm = W.reshape(d // 32, 16, 2, -1).transpose(0, 2, 1, 3).reshape(d, -1)
    partials = _sc_aggregate(feat_packed, eflat, wgt, n_pad)
    return _tc_combine_matmul_relu(partials, W_perm, n_nodes)


# speedup vs baseline: 2.1930x; 1.1262x over previous
import functools

import jax
import jax.numpy as jnp
from jax import lax
from jax.experimental import pallas as pl
from jax.experimental.pallas import tpu as pltpu
from jax.experimental.pallas import tpu_sc as plsc

NC = 2
NS = 16
NW = NC * NS
LANES = 16
CHUNK = 80
RING = 4


def _sc_aggregate(features, eflat, wgt, n_pad):
    n_nodes, d = features.shape
    n_edges = eflat.shape[0] // 2
    assert n_edges % NW == 0
    e_w = n_edges // NW
    assert e_w % CHUNK == 0
    n_chunks = e_w // CHUNK
    assert n_chunks >= RING
    n_iters = -(-(n_chunks + 2) // RING) * RING
    assert n_iters >= n_chunks + 2
    assert n_pad % (NS * 8) == 0
    rows_w = n_pad // NS
    assert rows_w % CHUNK == 0
    d_vecs = d // LANES
    groups = CHUNK // LANES

    mesh = plsc.VectorSubcoreMesh(core_axis_name="c", subcore_axis_name="s")

    @functools.partial(
        pl.kernel,
        out_type=jax.ShapeDtypeStruct((NC, n_pad, d), jnp.float32),
        mesh=mesh,
        scratch_types=[
            pltpu.VMEM((RING, CHUNK), jnp.int32),
            pltpu.VMEM((RING, CHUNK), jnp.int32),
            pltpu.VMEM((RING, CHUNK), jnp.float32),
            pltpu.VMEM((RING, CHUNK), jnp.int32),
            pltpu.VMEM((RING, CHUNK), jnp.float32),
            pltpu.VMEM((RING, CHUNK, d), jnp.float32),
            pltpu.VMEM_SHARED((n_pad, d), jnp.float32),
            [pltpu.SemaphoreType.DMA] * RING,
            [pltpu.SemaphoreType.DMA] * RING,
            [pltpu.SemaphoreType.DMA] * RING,
        ],
    )
    def agg(feat_hbm, eflat_hbm, wgt_hbm, out_hbm,
            src_f, dst_f, w_f, sdst, sw, rows, acc,
            sem_i, sem_g, sem_s):
        c = lax.axis_index("c")
        s = lax.axis_index("s")
        wid = c * NS + s
        e0 = wid * e_w

        def idx_start(i, r):
            base = e0 + i * CHUNK
            pltpu.async_copy(eflat_hbm.at[pl.ds(n_edges + base, CHUNK)], src_f.at[r], sem_i[r])
            pltpu.async_copy(eflat_hbm.at[pl.ds(base, CHUNK)], dst_f.at[r], sem_i[r])
            pltpu.async_copy(wgt_hbm.at[pl.ds(base, CHUNK)], w_f.at[r], sem_i[r])

        def idx_wait(r):
            pltpu.make_async_copy(eflat_hbm.at[pl.ds(0, CHUNK)], src_f.at[r], sem_i[r]).wait()
            pltpu.make_async_copy(eflat_hbm.at[pl.ds(0, CHUNK)], dst_f.at[r], sem_i[r]).wait()
            pltpu.make_async_copy(wgt_hbm.at[pl.ds(0, CHUNK)], w_f.at[r], sem_i[r]).wait()

        half = CHUNK // 2

        def gather_start(r):
            pltpu.async_copy(feat_hbm.at[src_f.at[r, pl.ds(0, half)]],
                             rows.at[r, pl.ds(0, half), :], sem_g[r])
            pltpu.async_copy(feat_hbm.at[src_f.at[r, pl.ds(half, half)]],
                             rows.at[r, pl.ds(half, half), :], sem_g[r])

        def gather_wait(r):
            pltpu.make_async_copy(feat_hbm.at[src_f.at[r, pl.ds(0, half)]],
                                  rows.at[r, pl.ds(0, half), :], sem_g[r]).wait()
            pltpu.make_async_copy(feat_hbm.at[src_f.at[r, pl.ds(half, half)]],
                                  rows.at[r, pl.ds(half, half), :], sem_g[r]).wait()

        def scat_start(r):
            pltpu.async_copy(rows.at[r], acc.at[sdst.at[r]], sem_s[r], add=True)

        def scat_wait(r):
            pltpu.make_async_copy(rows.at[r], acc.at[sdst.at[r]], sem_s[r]).wait()

        def stash_idx(r):
            for g in range(groups):
                sl = pl.ds(g * LANES, LANES)
                sdst[r, sl] = dst_f[r, sl]
                sw[r, sl] = w_f[r, sl]

        def compute(r):
            def group_body(g, carry):
                wv16 = sw[r, pl.ds(g * LANES, LANES)]
                for e16 in range(LANES):
                    wbc = lax.gather(
                        wv16,
                        jnp.full((LANES, 1), e16, jnp.int32),
                        dimension_numbers=lax.GatherDimensionNumbers(
                            offset_dims=(), collapsed_slice_dims=(0,),
                            start_index_map=(0,)),
                        slice_sizes=(1,),
                        mode=lax.GatherScatterMode.PROMISE_IN_BOUNDS)
                    e = g * LANES + e16
                    for j in range(d_vecs):
                        sl = pl.ds(j * LANES, LANES)
                        rows[r, e, sl] = rows[r, e, sl] * wbc
                return carry
            lax.fori_loop(0, groups, group_body, 0)

        idx_start(0, 0)
        idx_start(1, 1)
        idx_start(2, 2)

        def zfill(i, carry):
            for j in range(d_vecs):
                rows[0, i, pl.ds(j * LANES, LANES)] = jnp.zeros((LANES,), jnp.float32)
            return carry
        lax.fori_loop(0, CHUNK, zfill, 0)
        r0 = s * rows_w

        def zcopy(i, carry):
            pltpu.sync_copy(rows.at[0], acc.at[pl.ds(r0 + i * CHUNK, CHUNK), :])
            return carry
        lax.fori_loop(0, rows_w // CHUNK, zcopy, 0)

        idx_wait(0)
        gather_start(0)
        idx_wait(1)
        gather_start(1)
        plsc.subcore_barrier()

        def step(k, carry):
            for j in range(RING):
                i = k * RING + j
                live = i < n_chunks

                @pl.when(live)
                def _():
                    gather_wait(j)
                    stash_idx(j)

                @pl.when(i + 3 < n_chunks)
                def _():
                    idx_start(i + 3, (j + 3) % RING)

                @pl.when(jnp.logical_and(i >= 2, i - 2 < n_chunks))
                def _():
                    scat_wait((j + 2) % RING)

                @pl.when(i + 2 < n_chunks)
                def _():
                    idx_wait((j + 2) % RING)
                    gather_start((j + 2) % RING)

                @pl.when(live)
                def _():
                    compute(j)
                    scat_start(j)
            return carry
        lax.fori_loop(0, n_iters // RING, step, 0)
        plsc.subcore_barrier()

        pltpu.sync_copy(acc.at[pl.ds(r0, rows_w), :],
                        out_hbm.at[c, pl.ds(r0, rows_w), :])

    return agg(features, eflat, wgt)


def _tc_combine_matmul_relu(partials, W, n_nodes):
    _, n_pad, d = partials.shape
    d_out = W.shape[1]
    blk = 1000
    assert n_nodes % blk == 0

    def body(p_ref, w_ref, o_ref):
        pp = p_ref[0] + p_ref[1]
        acc = jnp.dot(pp, w_ref[...], preferred_element_type=jnp.float32,
                      precision=lax.Precision.HIGHEST)
        o_ref[...] = jnp.maximum(acc, 0.0)

    return pl.pallas_call(
        body,
        grid=(n_nodes // blk,),
        in_specs=[
            pl.BlockSpec((NC, blk, d), lambda i: (0, i, 0)),
            pl.BlockSpec((d, d_out), lambda i: (0, 0)),
        ],
        out_specs=pl.BlockSpec((blk, d_out), lambda i: (i, 0)),
        out_shape=jax.ShapeDtypeStruct((n_nodes, d_out), jnp.float32),
    )(partials, W)


def kernel(features, edge_index, edge_weight, W):
    n_nodes = features.shape[0]
    n_pad = 10240
    eflat = edge_index.astype(jnp.int32).reshape(-1)
    wgt = edge_weight.astype(jnp.float32)
    partials = _sc_aggregate(features, eflat, wgt, n_pad)
    return _tc_combine_matmul_relu(partials, W, n_nodes)
